# baseline (speedup 1.0000x reference)
import functools

import jax
import jax.numpy as jnp
from jax import lax
from jax.experimental import pallas as pl
from jax.experimental.pallas import tpu as pltpu
from jax.experimental.pallas import tpu_sc as plsc

_B, _F, _G, _S = 16384, 208, 26, 8
_NC, _NS, _L = 2, 16, 16
_NW = _NC * _NS
_RPW = _B // _NW
_CH = 128
_NCH = _RPW // _CH
_NJ = _F // _L
_GPAD = 32


def _sc_body(x_hbm, gidx_hbm, wg_hbm, wfc_hbm, out_hbm,
             xv0, xv1, gidx_v, wg_v, wfc_v, w_v, out_v, sem0, sem1):
    wid = lax.axis_index("s") * _NC + lax.axis_index("c")
    base = wid * _RPW
    xbufs = (xv0, xv1)
    sems = (sem0, sem1)

    first = pltpu.make_async_copy(x_hbm.at[pl.ds(base, _CH)], xbufs[0], sems[0])
    first.start()

    pltpu.sync_copy(gidx_hbm, gidx_v)
    pltpu.sync_copy(wg_hbm, wg_v)
    pltpu.sync_copy(wfc_hbm, wfc_v)

    for j in range(_NJ):
        w_v[pl.ds(j * _L, _L)] = jnp.zeros((_L,), jnp.float32)
    lane_lo = lax.iota(jnp.int32, _L) < _S
    wfc_a = wfc_v[pl.ds(0, _L)]
    wfc_b = wfc_v[pl.ds(_L, _L)]

    def _wfc_at(g):
        return wfc_a[g] if g < _L else wfc_b[g - _L]

    for j in range(_NJ):
        wfc_g = jnp.where(lane_lo,
                          jnp.full((_L,), _wfc_at(2 * j), jnp.float32),
                          jnp.full((_L,), _wfc_at(2 * j + 1), jnp.float32))
        prod = wg_v[pl.ds(j * _L, _L)] * wfc_g
        plsc.addupdate_scatter(w_v, [gidx_v[pl.ds(j * _L, _L)]], prod)

    wjs = [w_v[pl.ds(j * _L, _L)] for j in range(_NJ)]

    pending = first
    for ch in range(_NCH):
        pending.wait()
        if ch + 1 < _NCH:
            pending = pltpu.make_async_copy(
                x_hbm.at[pl.ds(base + (ch + 1) * _CH, _CH)],
                xbufs[(ch + 1) % 2], sems[(ch + 1) % 2])
            pending.start()
        xv = xbufs[ch % 2]

        lane0 = lax.iota(jnp.int32, _L) == 0

        def row_body(r, carry, xv=xv, off=ch * _CH):
            acc = xv[r, pl.ds(0, _L)] * wjs[0]
            for j in range(1, _NJ):
                acc = acc + xv[r, pl.ds(j * _L, _L)] * wjs[j]
            s = jnp.sum(acc)
            idxv = jnp.full((_L,), off + r, jnp.int32)
            plsc.store_scatter(out_v, [idxv], jnp.full((_L,), s, jnp.float32),
                               mask=lane0)
            return carry

        lax.fori_loop(0, _CH, row_body, 0, unroll=4)

    pltpu.sync_copy(out_v, out_hbm.at[pl.ds(base, _RPW)])


@functools.partial(jax.jit, static_argnames=())
def _sc_call(x, gidx, wg, wfc):
    mesh = plsc.VectorSubcoreMesh(core_axis_name="c", subcore_axis_name="s")
    return pl.kernel(
        _sc_body,
        out_type=jax.ShapeDtypeStruct((_B,), jnp.float32),
        mesh=mesh,
        scratch_types=[
            pltpu.VMEM((_CH, _F), jnp.float32),
            pltpu.VMEM((_CH, _F), jnp.float32),
            pltpu.VMEM((_F,), jnp.int32),
            pltpu.VMEM((_F,), jnp.float32),
            pltpu.VMEM((_GPAD,), jnp.float32),
            pltpu.VMEM((_F,), jnp.float32),
            pltpu.VMEM((_RPW,), jnp.float32),
            pltpu.SemaphoreType.DMA,
            pltpu.SemaphoreType.DMA,
        ],
        compiler_params=pltpu.CompilerParams(needs_layout_passes=False),
    )(x, gidx, wg, wfc)


def kernel(x, group_idx, W_g, W_fc):
    gidx = group_idx.reshape(_F).astype(jnp.int32)
    wg = W_g.reshape(_F)
    wfc = jnp.pad(W_fc.reshape(_G), (0, _GPAD - _G))
    out = _sc_call(x, gidx, wg, wfc)
    return out.reshape(_B, 1)


# speedup vs baseline: 1.0333x; 1.0333x over previous
import functools

import jax
import jax.numpy as jnp
from jax import lax
from jax.experimental import pallas as pl
from jax.experimental.pallas import tpu as pltpu
from jax.experimental.pallas import tpu_sc as plsc

_B, _F, _G, _S = 16384, 208, 26, 8
_NC, _NS, _L = 2, 16, 16
_NW = _NC * _NS
_RPW = _B // _NW
_CH = 128
_NCH = _RPW // _CH
_NJ = _F // _L
_GPAD = 32


def _sc_body(x_hbm, gidx_hbm, wg_hbm, wfc_hbm, out_hbm,
             xv0, xv1, gidx_v, wg_v, wfc_v, w_v, out_v, sem0, sem1):
    wid = lax.axis_index("s") * _NC + lax.axis_index("c")
    base = wid * _RPW
    xbufs = (xv0, xv1)
    sems = (sem0, sem1)

    first = pltpu.make_async_copy(x_hbm.at[pl.ds(base, _CH)], xbufs[0], sems[0])
    first.start()

    pltpu.sync_copy(gidx_hbm, gidx_v)
    pltpu.sync_copy(wg_hbm, wg_v)
    pltpu.sync_copy(wfc_hbm, wfc_v)

    for j in range(_NJ):
        w_v[pl.ds(j * _L, _L)] = jnp.zeros((_L,), jnp.float32)
    lane_lo = lax.iota(jnp.int32, _L) < _S
    wfc_a = wfc_v[pl.ds(0, _L)]
    wfc_b = wfc_v[pl.ds(_L, _L)]

    def _wfc_at(g):
        return wfc_a[g] if g < _L else wfc_b[g - _L]

    for j in range(_NJ):
        wfc_g = jnp.where(lane_lo,
                          jnp.full((_L,), _wfc_at(2 * j), jnp.float32),
                          jnp.full((_L,), _wfc_at(2 * j + 1), jnp.float32))
        prod = wg_v[pl.ds(j * _L, _L)] * wfc_g
        plsc.addupdate_scatter(w_v, [gidx_v[pl.ds(j * _L, _L)]], prod)

    wjs = [w_v[pl.ds(j * _L, _L)] for j in range(_NJ)]

    pending = first
    for ch in range(_NCH):
        pending.wait()
        if ch + 1 < _NCH:
            pending = pltpu.make_async_copy(
                x_hbm.at[pl.ds(base + (ch + 1) * _CH, _CH)],
                xbufs[(ch + 1) % 2], sems[(ch + 1) % 2])
            pending.start()
        xv = xbufs[ch % 2]

        lane0 = lax.iota(jnp.int32, _L) == 0

        def row_body(r, carry, xv=xv, off=ch * _CH):
            terms = [xv[r, pl.ds(j * _L, _L)] * wjs[j] for j in range(_NJ)]
            while len(terms) > 1:
                terms = [terms[i] + terms[i + 1]
                         for i in range(0, len(terms) - 1, 2)] + (
                             [terms[-1]] if len(terms) % 2 else [])
            s = jnp.sum(terms[0])
            idxv = jnp.full((_L,), off + r, jnp.int32)
            plsc.store_scatter(out_v, [idxv], jnp.full((_L,), s, jnp.float32),
                               mask=lane0)
            return carry

        lax.fori_loop(0, _CH, row_body, 0, unroll=8)

    pltpu.sync_copy(out_v, out_hbm.at[pl.ds(base, _RPW)])


@functools.partial(jax.jit, static_argnames=())
def _sc_call(x, gidx, wg, wfc):
    mesh = plsc.VectorSubcoreMesh(core_axis_name="c", subcore_axis_name="s")
    return pl.kernel(
        _sc_body,
        out_type=jax.ShapeDtypeStruct((_B,), jnp.float32),
        mesh=mesh,
        scratch_types=[
            pltpu.VMEM((_CH, _F), jnp.float32),
            pltpu.VMEM((_CH, _F), jnp.float32),
            pltpu.VMEM((_F,), jnp.int32),
            pltpu.VMEM((_F,), jnp.float32),
            pltpu.VMEM((_GPAD,), jnp.float32),
            pltpu.VMEM((_F,), jnp.float32),
            pltpu.VMEM((_RPW,), jnp.float32),
            pltpu.SemaphoreType.DMA,
            pltpu.SemaphoreType.DMA,
        ],
        compiler_params=pltpu.CompilerParams(needs_layout_passes=False),
    )(x, gidx, wg, wfc)


def kernel(x, group_idx, W_g, W_fc):
    gidx = group_idx.reshape(_F).astype(jnp.int32)
    wg = W_g.reshape(_F)
    wfc = jnp.pad(W_fc.reshape(_G), (0, _GPAD - _G))
    out = _sc_call(x, gidx, wg, wfc)
    return out.reshape(_B, 1)
